# baseline (speedup 1.0000x reference)
import functools

import jax
import jax.numpy as jnp
from jax.experimental import pallas as pl
from jax.experimental.pallas import tpu as pltpu

M = 3136
K = 64
N = 65536
FH = 56
IMG = 224

NB1 = 16384
NSTEPS1 = N // NB1
LW = 128
KA = K + 1
NB2 = 16384
NSTEPS2 = N // NB2


def _copy_aug(lib_ref, slot_ref):
    b = lib_ref[...]
    slot_ref[:, 0:K] = b
    slot_ref[:, K:KA] = jnp.sum(b * b, axis=1, keepdims=True)


def _dot_fold(a, slot_ref, acc_ref, first):
    CN = 512
    rms = []
    for c0 in range(0, NB1, CN):
        t = jax.lax.dot_general(
            a, slot_ref[pl.ds(c0, CN), :], (((1,), (1,)), ((), ())),
            preferred_element_type=jnp.float32)
        parts = [t[:, c:c + LW] for c in range(0, CN, LW)]
        while len(parts) > 1:
            parts = [jnp.minimum(parts[i], parts[i + 1])
                     for i in range(0, len(parts), 2)]
        rms.append(parts[0])
    while len(rms) > 1:
        rms = [jnp.minimum(rms[i], rms[i + 1]) for i in range(0, len(rms), 2)]
    rm = rms[0]

    @pl.when(first)
    def _():
        acc_ref[...] = rm

    @pl.when(jnp.logical_not(first))
    def _():
        acc_ref[...] = jnp.minimum(acc_ref[...], rm)


def _min_body(aug_ref, lib_ref, val_ref, sd2_ref, mtest_ref,
              acc_ref, slot0_ref, slot1_ref):
    j = pl.program_id(0)
    a = aug_ref[...]

    @pl.when(j % 2 == 0)
    def _():
        @pl.when(j < NSTEPS1)
        def _():
            _copy_aug(lib_ref, slot0_ref)

        @pl.when(j > 0)
        def _():
            _dot_fold(a, slot1_ref, acc_ref, j == 1)

    @pl.when(j % 2 == 1)
    def _():
        @pl.when(j < NSTEPS1)
        def _():
            _copy_aug(lib_ref, slot1_ref)

        @pl.when(j > 0)
        def _():
            _dot_fold(a, slot0_ref, acc_ref, j == 1)

    @pl.when(j == NSTEPS1)
    def _():
        ap = a[:, 0:K]
        a2 = 0.25 * jnp.sum(ap * ap, axis=1, keepdims=True)
        mn = jnp.min(acc_ref[...], axis=1, keepdims=True)
        v = jnp.maximum(a2 + mn, 0.0)
        val_ref[...] = v
        mx = jnp.max(v)
        row = jax.lax.broadcasted_iota(jnp.int32, (M, 1), 0)
        p = jnp.min(jnp.where(v == mx, row, M))
        sd2_ref[...] = mx.reshape(1, 1)
        mtest_ref[...] = -0.5 * aug_ref[pl.ds(p, 1), 0:K]


def _reweight_body(mimg_ref, amat_ref, sd2_ref, mtest_ref, lib_ref,
                   s_ref, smap_ref,
                   dall_ref, dtall_ref, cand_ref, bmin_ref, mstar_ref):
    j = pl.program_id(0)

    @pl.when(j == 0)
    def _():
        mv = jnp.sqrt(mimg_ref[...])
        amat = amat_ref[...]
        t1 = jnp.dot(amat, mv, preferred_element_type=jnp.float32)
        smap_ref[...] = jax.lax.dot_general(
            t1, amat, (((1,), (1,)), ((), ())),
            preferred_element_type=jnp.float32)

    b = lib_ref[...]
    ones = jnp.ones((1, K), dtype=jnp.float32)
    b2 = jax.lax.dot_general(ones, b * b, (((1,), (1,)), ((), ())),
                             preferred_element_type=jnp.float32)
    lane = jax.lax.broadcasted_iota(jnp.int32, (1, NB2), 1)

    @pl.when(j < NSTEPS2)
    def _():
        mt = mtest_ref[...]
        mt2 = jnp.sum(mt * mt, axis=1, keepdims=True)
        trow = jnp.maximum(mt2 + b2 - 2.0 * jax.lax.dot_general(
            mt, b, (((1,), (1,)), ((), ())),
            preferred_element_type=jnp.float32), 0.0)
        dtall_ref[pl.ds(j, 1), :] = trow
        m = jnp.min(trow)
        li = jnp.min(jnp.where(trow == m, lane, NB2))
        bmin_ref[pl.ds(j, 1), :] = m.reshape(1, 1)
        cand_ref[pl.ds(j, 1), :] = lib_ref[pl.ds(li, 1), :]

        @pl.when(j == NSTEPS2 - 1)
        def _():
            bm = bmin_ref[...]
            mn = jnp.min(bm)
            rio = jax.lax.broadcasted_iota(jnp.int32, (NSTEPS2, 1), 0)
            jb = jnp.min(jnp.where(bm == mn, rio, NSTEPS2))
            mstar_ref[...] = cand_ref[pl.ds(jb, 1), :]

    @pl.when(j >= NSTEPS2)
    def _():
        jj = j - NSTEPS2
        ms = mstar_ref[...]
        ms2 = jnp.sum(ms * ms, axis=1, keepdims=True)
        wrow = jnp.maximum(ms2 + b2 - 2.0 * jax.lax.dot_general(
            ms, b, (((1,), (1,)), ((), ())),
            preferred_element_type=jnp.float32), 0.0)
        dall_ref[pl.ds(jj, 1), :] = wrow

        @pl.when(j == 2 * NSTEPS2 - 1)
        def _():
            d = dall_ref[...]
            dt = dtall_ref[...]
            fio = (jax.lax.broadcasted_iota(jnp.int32, (NSTEPS2, NB2), 0) * NB2
                   + jax.lax.broadcasted_iota(jnp.int32, (NSTEPS2, NB2), 1))
            dts = []
            for _t in range(3):
                m = jnp.min(d)
                p = jnp.min(jnp.where(d == m, fio, N))
                dts.append(jnp.sum(jnp.where(fio == p, dt, 0.0)))
                d = jnp.where(fio == p, jnp.inf, d)
            sstar = jnp.sqrt(sd2_ref[...])
            dinv = 1.0 / jnp.sqrt(jnp.float32(K))
            mk1 = jnp.sqrt(dts[1])
            mk2 = jnp.sqrt(dts[2])
            w = 1.0 - jnp.exp(sstar * dinv) / (jnp.exp(mk1 * dinv)
                                               + jnp.exp(mk2 * dinv))
            s_ref[...] = w * sstar


@functools.lru_cache(maxsize=1)
def _blur_np():
    import numpy as np
    x = np.arange(-8, 9, dtype=np.float64)
    kk = np.exp(-0.5 * (x / 4.0) ** 2).astype(np.float32)
    kk = kk / kk.sum(dtype=np.float32)
    idx = np.arange(IMG)
    off = idx[None, :] - idx[:, None] + 8
    g = np.where((off >= 0) & (off <= 16), kk[np.clip(off, 0, 16)], 0.0)
    return np.asarray(g, np.float32)


def _build_amat():
    r = jax.image.resize(jnp.eye(FH, dtype=jnp.float32), (IMG, FH),
                         method='bilinear')
    return jnp.asarray(_blur_np()) @ r


def _min_pass(aug, patch_lib, interpret=False):
    return pl.pallas_call(
        _min_body,
        grid=(NSTEPS1 + 1,),
        in_specs=[
            pl.BlockSpec((M, KA), lambda j: (0, 0)),
            pl.BlockSpec((NB1, K), lambda j: (jnp.minimum(j, NSTEPS1 - 1), 0)),
        ],
        out_specs=[
            pl.BlockSpec((M, 1), lambda j: (0, 0)),
            pl.BlockSpec((1, 1), lambda j: (0, 0)),
            pl.BlockSpec((1, K), lambda j: (0, 0)),
        ],
        out_shape=[
            jax.ShapeDtypeStruct((M, 1), jnp.float32),
            jax.ShapeDtypeStruct((1, 1), jnp.float32),
            jax.ShapeDtypeStruct((1, K), jnp.float32),
        ],
        scratch_shapes=[pltpu.VMEM((M, LW), jnp.float32),
                        pltpu.VMEM((NB1, KA), jnp.float32),
                        pltpu.VMEM((NB1, KA), jnp.float32)],
        compiler_params=pltpu.CompilerParams(
            dimension_semantics=("arbitrary",)),
        interpret=interpret,
    )(aug, patch_lib)


def _reweight_pass(mimg, amat, sd2, mtest, patch_lib, interpret=False):
    return pl.pallas_call(
        _reweight_body,
        grid=(2 * NSTEPS2,),
        in_specs=[
            pl.BlockSpec((FH, FH), lambda j: (0, 0)),
            pl.BlockSpec((IMG, FH), lambda j: (0, 0)),
            pl.BlockSpec((1, 1), lambda j: (0, 0)),
            pl.BlockSpec((1, K), lambda j: (0, 0)),
            pl.BlockSpec((NB2, K), lambda j: (j % NSTEPS2, 0)),
        ],
        out_specs=[
            pl.BlockSpec((1, 1), lambda j: (0, 0)),
            pl.BlockSpec((IMG, IMG), lambda j: (0, 0)),
        ],
        out_shape=[
            jax.ShapeDtypeStruct((1, 1), jnp.float32),
            jax.ShapeDtypeStruct((IMG, IMG), jnp.float32),
        ],
        scratch_shapes=[
            pltpu.VMEM((NSTEPS2, NB2), jnp.float32),
            pltpu.VMEM((NSTEPS2, NB2), jnp.float32),
            pltpu.VMEM((NSTEPS2, K), jnp.float32),
            pltpu.VMEM((NSTEPS2, 1), jnp.float32),
            pltpu.VMEM((1, K), jnp.float32),
        ],
        compiler_params=pltpu.CompilerParams(
            dimension_semantics=("arbitrary",)),
        interpret=interpret,
    )(mimg, amat, sd2, mtest, patch_lib)


def kernel(patch, patch_lib):
    aug = jnp.concatenate(
        [patch * (-2.0), jnp.ones((M, 1), jnp.float32)], axis=1)
    mind2, sd2, mtest = _min_pass(aug, patch_lib)
    amat = _build_amat()
    s11, smap = _reweight_pass(mind2.reshape(FH, FH), amat, sd2, mtest,
                               patch_lib)
    return s11.reshape(()), smap.reshape(1, 1, IMG, IMG)


# speedup vs baseline: 1.7493x; 1.7493x over previous
import functools

import jax
import jax.numpy as jnp
from jax.experimental import pallas as pl
from jax.experimental.pallas import tpu as pltpu

M = 3136
K = 64
N = 65536
FH = 56
IMG = 224

NB1 = 16384
NSTEPS1 = N // NB1
LW = 128
KA = K + 1
NB2 = 16384
NSTEPS2 = N // NB2


def _min_body(aug_ref, lib_ref, val_ref, sd2_ref, mtest_ref, acc_ref):
    j = pl.program_id(0)
    a = -2.0 * aug_ref[...]
    b = lib_ref[...]
    ones = jnp.ones((1, K), dtype=jnp.float32)
    b2 = jax.lax.dot_general(ones, b * b, (((1,), (1,)), ((), ())),
                             preferred_element_type=jnp.float32)
    CN = 512
    rms = []
    for c0 in range(0, NB1, CN):
        t = jax.lax.dot_general(
            a, b[c0:c0 + CN, :], (((1,), (1,)), ((), ())),
            preferred_element_type=jnp.float32)
        parts = [t[:, c:c + LW] + b2[:, c0 + c:c0 + c + LW]
                 for c in range(0, CN, LW)]
        while len(parts) > 1:
            parts = [jnp.minimum(parts[i], parts[i + 1])
                     for i in range(0, len(parts), 2)]
        rms.append(parts[0])
    while len(rms) > 1:
        rms = [jnp.minimum(rms[i], rms[i + 1]) for i in range(0, len(rms), 2)]
    rm = rms[0]

    @pl.when(j == 0)
    def _():
        acc_ref[...] = rm

    @pl.when(j > 0)
    def _():
        acc_ref[...] = jnp.minimum(acc_ref[...], rm)

    @pl.when(j == NSTEPS1 - 1)
    def _():
        ap = aug_ref[...]
        a2 = jnp.sum(ap * ap, axis=1, keepdims=True)
        mn = jnp.min(acc_ref[...], axis=1, keepdims=True)
        v = jnp.maximum(a2 + mn, 0.0)
        val_ref[...] = v
        mx = jnp.max(v)
        row = jax.lax.broadcasted_iota(jnp.int32, (M, 1), 0)
        p = jnp.min(jnp.where(v == mx, row, M))
        sd2_ref[...] = mx.reshape(1, 1)
        mtest_ref[...] = aug_ref[pl.ds(p, 1), :]


def _reweight_body(mimg_ref, amat_ref, sd2_ref, mtest_ref, lib_ref,
                   s_ref, smap_ref,
                   dall_ref, dtall_ref, cand_ref, bmin_ref, mstar_ref,
                   b2all_ref):
    j = pl.program_id(0)

    @pl.when(j == 0)
    def _():
        mv = jnp.sqrt(mimg_ref[...])
        amat = amat_ref[...]
        t1 = jnp.dot(amat, mv, preferred_element_type=jnp.float32)
        smap_ref[...] = jax.lax.dot_general(
            t1, amat, (((1,), (1,)), ((), ())),
            preferred_element_type=jnp.float32)

    b = lib_ref[...]
    lane = jax.lax.broadcasted_iota(jnp.int32, (1, NB2), 1)

    @pl.when(j < NSTEPS2)
    def _():
        ones = jnp.ones((1, K), dtype=jnp.float32)
        b2 = jax.lax.dot_general(ones, b * b, (((1,), (1,)), ((), ())),
                                 preferred_element_type=jnp.float32)
        b2all_ref[pl.ds(j, 1), :] = b2
        mt = mtest_ref[...]
        mt2 = jnp.sum(mt * mt, axis=1, keepdims=True)
        trow = jnp.maximum(mt2 + b2 - 2.0 * jax.lax.dot_general(
            mt, b, (((1,), (1,)), ((), ())),
            preferred_element_type=jnp.float32), 0.0)
        dtall_ref[pl.ds(j, 1), :] = trow
        m = jnp.min(trow)
        li = jnp.min(jnp.where(trow == m, lane, NB2))
        bmin_ref[pl.ds(j, 1), :] = m.reshape(1, 1)
        cand_ref[pl.ds(j, 1), :] = lib_ref[pl.ds(li, 1), :]

        @pl.when(j == NSTEPS2 - 1)
        def _():
            bm = bmin_ref[...]
            mn = jnp.min(bm)
            rio = jax.lax.broadcasted_iota(jnp.int32, (NSTEPS2, 1), 0)
            jb = jnp.min(jnp.where(bm == mn, rio, NSTEPS2))
            mstar_ref[...] = cand_ref[pl.ds(jb, 1), :]

    @pl.when(j >= NSTEPS2)
    def _():
        jj = j - NSTEPS2
        b2 = b2all_ref[pl.ds(jj, 1), :]
        ms = mstar_ref[...]
        ms2 = jnp.sum(ms * ms, axis=1, keepdims=True)
        wrow = jnp.maximum(ms2 + b2 - 2.0 * jax.lax.dot_general(
            ms, b, (((1,), (1,)), ((), ())),
            preferred_element_type=jnp.float32), 0.0)
        dall_ref[pl.ds(jj, 1), :] = wrow

        @pl.when(j == 2 * NSTEPS2 - 1)
        def _():
            d = dall_ref[...]
            dt = dtall_ref[...]
            fio = (jax.lax.broadcasted_iota(jnp.int32, (NSTEPS2, NB2), 0) * NB2
                   + jax.lax.broadcasted_iota(jnp.int32, (NSTEPS2, NB2), 1))
            dts = []
            for _t in range(3):
                m = jnp.min(d)
                p = jnp.min(jnp.where(d == m, fio, N))
                dts.append(jnp.sum(jnp.where(fio == p, dt, 0.0)))
                d = jnp.where(fio == p, jnp.inf, d)
            sstar = jnp.sqrt(sd2_ref[...])
            dinv = 1.0 / jnp.sqrt(jnp.float32(K))
            mk1 = jnp.sqrt(dts[1])
            mk2 = jnp.sqrt(dts[2])
            w = 1.0 - jnp.exp(sstar * dinv) / (jnp.exp(mk1 * dinv)
                                               + jnp.exp(mk2 * dinv))
            s_ref[...] = w * sstar


@functools.lru_cache(maxsize=1)
def _blur_np():
    import numpy as np
    x = np.arange(-8, 9, dtype=np.float64)
    kk = np.exp(-0.5 * (x / 4.0) ** 2).astype(np.float32)
    kk = kk / kk.sum(dtype=np.float32)
    idx = np.arange(IMG)
    off = idx[None, :] - idx[:, None] + 8
    g = np.where((off >= 0) & (off <= 16), kk[np.clip(off, 0, 16)], 0.0)
    return np.asarray(g, np.float32)


def _build_amat():
    r = jax.image.resize(jnp.eye(FH, dtype=jnp.float32), (IMG, FH),
                         method='bilinear')
    return jnp.asarray(_blur_np()) @ r


def _min_pass(aug, patch_lib, interpret=False):
    return pl.pallas_call(
        _min_body,
        grid=(NSTEPS1,),
        in_specs=[
            pl.BlockSpec((M, K), lambda j: (0, 0)),
            pl.BlockSpec((NB1, K), lambda j: (j, 0)),
        ],
        out_specs=[
            pl.BlockSpec((M, 1), lambda j: (0, 0)),
            pl.BlockSpec((1, 1), lambda j: (0, 0)),
            pl.BlockSpec((1, K), lambda j: (0, 0)),
        ],
        out_shape=[
            jax.ShapeDtypeStruct((M, 1), jnp.float32),
            jax.ShapeDtypeStruct((1, 1), jnp.float32),
            jax.ShapeDtypeStruct((1, K), jnp.float32),
        ],
        scratch_shapes=[pltpu.VMEM((M, LW), jnp.float32)],
        compiler_params=pltpu.CompilerParams(
            dimension_semantics=("arbitrary",)),
        interpret=interpret,
    )(aug, patch_lib)


def _reweight_pass(mimg, amat, sd2, mtest, patch_lib, interpret=False):
    return pl.pallas_call(
        _reweight_body,
        grid=(2 * NSTEPS2,),
        in_specs=[
            pl.BlockSpec((FH, FH), lambda j: (0, 0)),
            pl.BlockSpec((IMG, FH), lambda j: (0, 0)),
            pl.BlockSpec((1, 1), lambda j: (0, 0)),
            pl.BlockSpec((1, K), lambda j: (0, 0)),
            pl.BlockSpec((NB2, K), lambda j: (j % NSTEPS2, 0)),
        ],
        out_specs=[
            pl.BlockSpec((1, 1), lambda j: (0, 0)),
            pl.BlockSpec((IMG, IMG), lambda j: (0, 0)),
        ],
        out_shape=[
            jax.ShapeDtypeStruct((1, 1), jnp.float32),
            jax.ShapeDtypeStruct((IMG, IMG), jnp.float32),
        ],
        scratch_shapes=[
            pltpu.VMEM((NSTEPS2, NB2), jnp.float32),
            pltpu.VMEM((NSTEPS2, NB2), jnp.float32),
            pltpu.VMEM((NSTEPS2, K), jnp.float32),
            pltpu.VMEM((NSTEPS2, 1), jnp.float32),
            pltpu.VMEM((1, K), jnp.float32),
            pltpu.VMEM((NSTEPS2, NB2), jnp.float32),
        ],
        compiler_params=pltpu.CompilerParams(
            dimension_semantics=("arbitrary",)),
        interpret=interpret,
    )(mimg, amat, sd2, mtest, patch_lib)


def kernel(patch, patch_lib):
    mind2, sd2, mtest = _min_pass(patch, patch_lib)
    amat = _build_amat()
    s11, smap = _reweight_pass(mind2.reshape(FH, FH), amat, sd2, mtest,
                               patch_lib)
    return s11.reshape(()), smap.reshape(1, 1, IMG, IMG)


# speedup vs baseline: 1.8037x; 1.0311x over previous
import functools

import jax
import jax.numpy as jnp
from jax.experimental import pallas as pl
from jax.experimental.pallas import tpu as pltpu

M = 3136
K = 64
N = 65536
FH = 56
IMG = 224

NB1 = 16384
NSTEPS1 = N // NB1
LW = 128
KA = K + 1
NB2 = 16384
NSTEPS2 = N // NB2


def _min_body(aug_ref, lib_ref, val_ref, sd2_ref, mtest_ref, acc_ref):
    j = pl.program_id(0)
    a = aug_ref[...]
    b = lib_ref[...]
    ones = jnp.ones((1, K), dtype=jnp.float32)
    b2 = jax.lax.dot_general(ones, b * b, (((1,), (1,)), ((), ())),
                             preferred_element_type=jnp.float32)
    CN = 512
    rms = []
    for c0 in range(0, NB1, CN):
        t = jax.lax.dot_general(
            a, b[c0:c0 + CN, :], (((1,), (1,)), ((), ())),
            preferred_element_type=jnp.float32)
        parts = [t[:, c:c + LW] + b2[:, c0 + c:c0 + c + LW]
                 for c in range(0, CN, LW)]
        while len(parts) > 1:
            parts = [jnp.minimum(parts[i], parts[i + 1])
                     for i in range(0, len(parts), 2)]
        rms.append(parts[0])
    while len(rms) > 1:
        rms = [jnp.minimum(rms[i], rms[i + 1]) for i in range(0, len(rms), 2)]
    rm = rms[0]

    @pl.when(j == 0)
    def _():
        acc_ref[...] = rm

    @pl.when(j > 0)
    def _():
        acc_ref[...] = jnp.minimum(acc_ref[...], rm)

    @pl.when(j == NSTEPS1 - 1)
    def _():
        a2 = 0.25 * jnp.sum(a * a, axis=1, keepdims=True)
        mn = jnp.min(acc_ref[...], axis=1, keepdims=True)
        v = jnp.maximum(a2 + mn, 0.0)
        val_ref[...] = v
        mx = jnp.max(v)
        row = jax.lax.broadcasted_iota(jnp.int32, (M, 1), 0)
        p = jnp.min(jnp.where(v == mx, row, M))
        sd2_ref[...] = mx.reshape(1, 1)
        mtest_ref[...] = -0.5 * aug_ref[pl.ds(p, 1), :]


def _reweight_body(mimg_ref, amat_ref, sd2_ref, mtest_ref, lib_ref,
                   s_ref, smap_ref,
                   dall_ref, dtall_ref, cand_ref, bmin_ref, mstar_ref):
    j = pl.program_id(0)

    @pl.when(j == 0)
    def _():
        mv = jnp.sqrt(mimg_ref[...])
        amat = amat_ref[...]
        t1 = jnp.dot(amat, mv, preferred_element_type=jnp.float32)
        smap_ref[...] = jax.lax.dot_general(
            t1, amat, (((1,), (1,)), ((), ())),
            preferred_element_type=jnp.float32)

    b = lib_ref[...]
    ones = jnp.ones((1, K), dtype=jnp.float32)
    b2 = jax.lax.dot_general(ones, b * b, (((1,), (1,)), ((), ())),
                             preferred_element_type=jnp.float32)
    lane = jax.lax.broadcasted_iota(jnp.int32, (1, NB2), 1)

    @pl.when(j < NSTEPS2)
    def _():
        mt = mtest_ref[...]
        mt2 = jnp.sum(mt * mt, axis=1, keepdims=True)
        trow = jnp.maximum(mt2 + b2 - 2.0 * jax.lax.dot_general(
            mt, b, (((1,), (1,)), ((), ())),
            preferred_element_type=jnp.float32), 0.0)
        dtall_ref[pl.ds(j, 1), :] = trow
        m = jnp.min(trow)
        li = jnp.min(jnp.where(trow == m, lane, NB2))
        bmin_ref[pl.ds(j, 1), :] = m.reshape(1, 1)
        cand_ref[pl.ds(j, 1), :] = lib_ref[pl.ds(li, 1), :]

        @pl.when(j == NSTEPS2 - 1)
        def _():
            bm = bmin_ref[...]
            mn = jnp.min(bm)
            rio = jax.lax.broadcasted_iota(jnp.int32, (NSTEPS2, 1), 0)
            jb = jnp.min(jnp.where(bm == mn, rio, NSTEPS2))
            mstar_ref[...] = cand_ref[pl.ds(jb, 1), :]

    @pl.when(j >= NSTEPS2)
    def _():
        jj = j - NSTEPS2
        ms = mstar_ref[...]
        ms2 = jnp.sum(ms * ms, axis=1, keepdims=True)
        wrow = jnp.maximum(ms2 + b2 - 2.0 * jax.lax.dot_general(
            ms, b, (((1,), (1,)), ((), ())),
            preferred_element_type=jnp.float32), 0.0)
        dall_ref[pl.ds(jj, 1), :] = wrow

        @pl.when(j == 2 * NSTEPS2 - 1)
        def _():
            d = dall_ref[...]
            dt = dtall_ref[...]
            fio = (jax.lax.broadcasted_iota(jnp.int32, (NSTEPS2, NB2), 0) * NB2
                   + jax.lax.broadcasted_iota(jnp.int32, (NSTEPS2, NB2), 1))
            dts = []
            for _t in range(3):
                m = jnp.min(d)
                p = jnp.min(jnp.where(d == m, fio, N))
                dts.append(jnp.sum(jnp.where(fio == p, dt, 0.0)))
                d = jnp.where(fio == p, jnp.inf, d)
            sstar = jnp.sqrt(sd2_ref[...])
            dinv = 1.0 / jnp.sqrt(jnp.float32(K))
            mk1 = jnp.sqrt(dts[1])
            mk2 = jnp.sqrt(dts[2])
            w = 1.0 - jnp.exp(sstar * dinv) / (jnp.exp(mk1 * dinv)
                                               + jnp.exp(mk2 * dinv))
            s_ref[...] = w * sstar


@functools.lru_cache(maxsize=1)
def _blur_np():
    import numpy as np
    x = np.arange(-8, 9, dtype=np.float64)
    kk = np.exp(-0.5 * (x / 4.0) ** 2).astype(np.float32)
    kk = kk / kk.sum(dtype=np.float32)
    idx = np.arange(IMG)
    off = idx[None, :] - idx[:, None] + 8
    g = np.where((off >= 0) & (off <= 16), kk[np.clip(off, 0, 16)], 0.0)
    return np.asarray(g, np.float32)


def _build_amat():
    r = jax.image.resize(jnp.eye(FH, dtype=jnp.float32), (IMG, FH),
                         method='bilinear')
    return jnp.asarray(_blur_np()) @ r


def _min_pass(aug, patch_lib, interpret=False):
    return pl.pallas_call(
        _min_body,
        grid=(NSTEPS1,),
        in_specs=[
            pl.BlockSpec((M, K), lambda j: (0, 0)),
            pl.BlockSpec((NB1, K), lambda j: (j, 0)),
        ],
        out_specs=[
            pl.BlockSpec((M, 1), lambda j: (0, 0)),
            pl.BlockSpec((1, 1), lambda j: (0, 0)),
            pl.BlockSpec((1, K), lambda j: (0, 0)),
        ],
        out_shape=[
            jax.ShapeDtypeStruct((M, 1), jnp.float32),
            jax.ShapeDtypeStruct((1, 1), jnp.float32),
            jax.ShapeDtypeStruct((1, K), jnp.float32),
        ],
        scratch_shapes=[pltpu.VMEM((M, LW), jnp.float32)],
        compiler_params=pltpu.CompilerParams(
            dimension_semantics=("arbitrary",)),
        interpret=interpret,
    )(aug, patch_lib)


def _reweight_pass(mimg, amat, sd2, mtest, patch_lib, interpret=False):
    return pl.pallas_call(
        _reweight_body,
        grid=(2 * NSTEPS2,),
        in_specs=[
            pl.BlockSpec((FH, FH), lambda j: (0, 0)),
            pl.BlockSpec((IMG, FH), lambda j: (0, 0)),
            pl.BlockSpec((1, 1), lambda j: (0, 0)),
            pl.BlockSpec((1, K), lambda j: (0, 0)),
            pl.BlockSpec((NB2, K), lambda j: (j % NSTEPS2, 0)),
        ],
        out_specs=[
            pl.BlockSpec((1, 1), lambda j: (0, 0)),
            pl.BlockSpec((IMG, IMG), lambda j: (0, 0)),
        ],
        out_shape=[
            jax.ShapeDtypeStruct((1, 1), jnp.float32),
            jax.ShapeDtypeStruct((IMG, IMG), jnp.float32),
        ],
        scratch_shapes=[
            pltpu.VMEM((NSTEPS2, NB2), jnp.float32),
            pltpu.VMEM((NSTEPS2, NB2), jnp.float32),
            pltpu.VMEM((NSTEPS2, K), jnp.float32),
            pltpu.VMEM((NSTEPS2, 1), jnp.float32),
            pltpu.VMEM((1, K), jnp.float32),
        ],
        compiler_params=pltpu.CompilerParams(
            dimension_semantics=("arbitrary",)),
        interpret=interpret,
    )(mimg, amat, sd2, mtest, patch_lib)


def kernel(patch, patch_lib):
    mind2, sd2, mtest = _min_pass(patch * (-2.0), patch_lib)
    amat = _build_amat()
    s11, smap = _reweight_pass(mind2.reshape(FH, FH), amat, sd2, mtest,
                               patch_lib)
    return s11.reshape(()), smap.reshape(1, 1, IMG, IMG)
